# baseline (speedup 1.0000x reference)
import functools

import jax
import jax.numpy as jnp
from jax import lax
from jax.experimental import pallas as pl
from jax.experimental.pallas import tpu as pltpu
from jax.experimental.pallas import tpu_sc as plsc

NUM_TOKENS = 1000000
DIM_MODEL = 64
BATCH = 4096
SEQ = 200

NC = 2
NS = 16
NW = NC * NS

N = BATCH * SEQ
N_PER_W = N // NW
CHUNK = 128
N_CHUNKS = N_PER_W // CHUNK


def _gather_sc(table, idx):
    mesh = plsc.VectorSubcoreMesh(core_axis_name="c", subcore_axis_name="s")

    @functools.partial(
        pl.kernel,
        mesh=mesh,
        out_type=jax.ShapeDtypeStruct((N, DIM_MODEL), jnp.float32),
        scratch_types=[
            pltpu.VMEM((N_PER_W,), jnp.int32),
            pltpu.VMEM((CHUNK, DIM_MODEL), jnp.float32),
            pltpu.SemaphoreType.DMA,
        ],
        compiler_params=pltpu.CompilerParams(use_tc_tiling_on_sc=False),
    )
    def k(table_hbm, idx_hbm, out_hbm, idx_v, rows_v, sem):
        wid = lax.axis_index("s") * NC + lax.axis_index("c")
        base = wid * N_PER_W
        pltpu.sync_copy(idx_hbm.at[pl.ds(base, N_PER_W)], idx_v)

        def step(j, carry):
            off = j * CHUNK
            pltpu.async_copy(
                table_hbm.at[idx_v.at[pl.ds(off, CHUNK)]], rows_v, sem
            ).wait()
            pltpu.sync_copy(rows_v, out_hbm.at[pl.ds(base + off, CHUNK)])
            return carry

        lax.fori_loop(0, N_CHUNKS, step, 0)

    return k(table, idx)


def kernel(src, embedding):
    idx = src.reshape(-1).astype(jnp.int32)
    out = _gather_sc(embedding, idx)
    return out.reshape(src.shape[0], src.shape[1], DIM_MODEL)


# speedup vs baseline: 1.1169x; 1.1169x over previous
import functools

import jax
import jax.numpy as jnp
from jax import lax
from jax.experimental import pallas as pl
from jax.experimental.pallas import tpu as pltpu
from jax.experimental.pallas import tpu_sc as plsc

NUM_TOKENS = 1000000
DIM_MODEL = 64
BATCH = 4096
SEQ = 200

NC = 2
NS = 16
NW = NC * NS

N = BATCH * SEQ
N_PER_W = N // NW
CHUNK = 128
K = 2
GROUP = CHUNK * K
GROUPS = N_PER_W // GROUP
NSLOT = 4


def _gather_sc(table, idx):
    mesh = plsc.VectorSubcoreMesh(core_axis_name="c", subcore_axis_name="s")

    @functools.partial(
        pl.kernel,
        mesh=mesh,
        out_type=jax.ShapeDtypeStruct((N, DIM_MODEL), jnp.float32),
        scratch_types=[
            pltpu.VMEM((N_PER_W,), jnp.int32),
            pltpu.VMEM((NSLOT, GROUP, DIM_MODEL), jnp.float32),
            [pltpu.SemaphoreType.DMA] * NSLOT,
            [pltpu.SemaphoreType.DMA] * NSLOT,
        ],
        compiler_params=pltpu.CompilerParams(use_tc_tiling_on_sc=False),
    )
    def k(table_hbm, idx_hbm, out_hbm, idx_v, rows_v, gsems, ssems):
        wid = lax.axis_index("s") * NC + lax.axis_index("c")
        base = wid * N_PER_W
        pltpu.sync_copy(idx_hbm.at[pl.ds(base, N_PER_W)], idx_v)

        def issue_gathers(g, s):
            off = g * GROUP
            for i in range(K):
                pltpu.async_copy(
                    table_hbm.at[idx_v.at[pl.ds(off + i * CHUNK, CHUNK)]],
                    rows_v.at[s, pl.ds(i * CHUNK, CHUNK)],
                    gsems[s],
                )

        def wait_gathers(g, s):
            off = g * GROUP
            for i in range(K):
                pltpu.make_async_copy(
                    table_hbm.at[idx_v.at[pl.ds(off + i * CHUNK, CHUNK)]],
                    rows_v.at[s, pl.ds(i * CHUNK, CHUNK)],
                    gsems[s],
                ).wait()

        def issue_store(g, s):
            pltpu.async_copy(
                rows_v.at[s], out_hbm.at[pl.ds(base + g * GROUP, GROUP)], ssems[s]
            )

        def wait_store(g, s):
            pltpu.make_async_copy(
                rows_v.at[s], out_hbm.at[pl.ds(base + g * GROUP, GROUP)], ssems[s]
            ).wait()

        issue_gathers(0, 0)
        issue_gathers(1, 1)
        for g in (0, 1):
            wait_gathers(g, g % NSLOT)
            issue_store(g, g % NSLOT)
            issue_gathers(g + 2, (g + 2) % NSLOT)
        for g in (2, 3):
            wait_gathers(g, g % NSLOT)
            issue_store(g, g % NSLOT)
            wait_store(g - 2, (g - 2) % NSLOT)
            issue_gathers(g + 2, (g + 2) % NSLOT)

        def quad_body(gq, carry):
            for h in range(NSLOT):
                g = NSLOT * gq + h
                wait_gathers(g, h)
                issue_store(g, h)
                wait_store(g - 2, (h - 2) % NSLOT)
                issue_gathers(g + 2, (h + 2) % NSLOT)
            return carry

        lax.fori_loop(1, GROUPS // NSLOT - 1, quad_body, 0)

        for g in range(GROUPS - NSLOT, GROUPS):
            s = g % NSLOT
            wait_gathers(g, s)
            issue_store(g, s)
            wait_store(g - 2, (g - 2) % NSLOT)
            if g + 2 < GROUPS:
                issue_gathers(g + 2, (g + 2) % NSLOT)
        for g in (GROUPS - 2, GROUPS - 1):
            wait_store(g, g % NSLOT)

    return k(table, idx)


def kernel(src, embedding):
    idx = src.reshape(-1).astype(jnp.int32)
    out = _gather_sc(embedding, idx)
    return out.reshape(src.shape[0], src.shape[1], DIM_MODEL)
